# baseline (speedup 1.0000x reference)
import functools

import jax
import jax.numpy as jnp
from jax import lax
from jax.experimental import pallas as pl
from jax.experimental.pallas import tpu as pltpu
from jax.experimental.pallas import tpu_sc as plsc

_NC, _NS, _L = 2, 16, 16
_NW = _NC * _NS
_E = 320000
_EPW = _E // _NW
_CH = 80
_NCHUNK = _EPW // _CH
_CAMS, _PTS, _D = 2000, 8000, 128
_F32 = jnp.float32

_mesh = plsc.VectorSubcoreMesh(
    core_axis_name="c", subcore_axis_name="s", num_cores=_NC, num_subcores=_NS
)


_NBUF = 5
_NBA = 4

@functools.partial(
    pl.kernel,
    out_type=(
        jax.ShapeDtypeStruct((_NC, _CAMS, _D), _F32),
        jax.ShapeDtypeStruct((_NC, _PTS, _D), _F32),
    ),
    mesh=_mesh,
    compiler_params=pltpu.CompilerParams(needs_layout_passes=False),
    scratch_types=(
        [pltpu.VMEM((_CH, _D), _F32)] * _NBA
        + [pltpu.VMEM((_CH,), jnp.int32)] * (2 * _NBA)
        + [pltpu.VMEM((16, _D), _F32),
           pltpu.VMEM_SHARED((_CAMS, _D), _F32),
           pltpu.VMEM_SHARED((_PTS, _D), _F32)]
        + [pltpu.SemaphoreType.DMA] * (2 * _NBA)
    ),
)
def _seg_sum(edges, ci, pi, cam_out, pt_out, *refs):
    rows = refs[0:_NBA]
    civ = refs[_NBA:2 * _NBA]
    piv = refs[2 * _NBA:3 * _NBA]
    zbuf, cam_sh, pt_sh = refs[3 * _NBA:3 * _NBA + 3]
    sem_in = refs[3 * _NBA + 3:4 * _NBA + 3]
    sem_sc = refs[4 * _NBA + 3:5 * _NBA + 3]
    c = lax.axis_index("c")
    s = lax.axis_index("s")
    zeros16 = jnp.zeros((_L,), _F32)

    wid = c * _NS + s
    base0 = wid * _EPW

    def start_in(b, k):
        base = base0 + k * _CH
        pltpu.async_copy(edges.at[pl.ds(base, _CH)], rows[b], sem_in[b])
        pltpu.async_copy(ci.at[pl.ds(base, _CH)], civ[b], sem_in[b])
        pltpu.async_copy(pi.at[pl.ds(base, _CH)], piv[b], sem_in[b])

    def wait_in(b):
        pltpu.make_async_copy(edges.at[pl.ds(0, _CH)], rows[b], sem_in[b]).wait()
        pltpu.make_async_copy(ci.at[pl.ds(0, _CH)], civ[b], sem_in[b]).wait()
        pltpu.make_async_copy(pi.at[pl.ds(0, _CH)], piv[b], sem_in[b]).wait()

    def wait_scat(b):
        pltpu.make_async_copy(rows[b], cam_sh.at[civ[b]], sem_sc[b]).wait()
        pltpu.make_async_copy(rows[b], pt_sh.at[piv[b]], sem_sc[b]).wait()

    for b in range(_NBA):
        start_in(b, b)

    @pl.loop(0, 16)
    def _zero(i):
        for j in range(_D // _L):
            zbuf[i, pl.ds(j * _L, _L)] = zeros16

    @pl.loop(0, _CAMS // 16)
    def _zc(b):
        @pl.when(lax.rem(b, _NS) == s)
        def _():
            pltpu.sync_copy(zbuf, cam_sh.at[pl.ds(b * 16, 16)])

    @pl.loop(0, _PTS // 16)
    def _zp(b):
        @pl.when(lax.rem(b, _NS) == s)
        def _():
            pltpu.sync_copy(zbuf, pt_sh.at[pl.ds(b * 16, 16)])

    plsc.subcore_barrier()

    def visit(k, b):
        wait_in(b)
        pltpu.async_copy(rows[b], cam_sh.at[civ[b]], sem_sc[b], add=True)
        pltpu.async_copy(rows[b], pt_sh.at[piv[b]], sem_sc[b], add=True)
        pb = (b - 1) % _NBA

        @pl.when(k > 0)
        def _():
            wait_scat(pb)

            @pl.when(k - 1 + _NBA < _NCHUNK)
            def _():
                start_in(pb, k - 1 + _NBA)

    @pl.loop(0, _NCHUNK - 1, step=_NBA)
    def _chunk(k0):
        for b in range(_NBA):
            visit(k0 + b, b)

    visit(_NCHUNK - 1, (_NCHUNK - 1) % _NBA)
    wait_scat((_NCHUNK - 1) % _NBA)
    plsc.subcore_barrier()

    @pl.loop(0, _CAMS // 16)
    def _wc(b):
        @pl.when(lax.rem(b, _NS) == s)
        def _():
            pltpu.sync_copy(cam_sh.at[pl.ds(b * 16, 16)],
                            cam_out.at[c, pl.ds(b * 16, 16)])

    @pl.loop(0, _PTS // 16)
    def _wp(b):
        @pl.when(lax.rem(b, _NS) == s)
        def _():
            pltpu.sync_copy(pt_sh.at[pl.ds(b * 16, 16)],
                            pt_out.at[c, pl.ds(b * 16, 16)])


def _proj_body(cam_parts, pt_parts, w_cam, w_point, w_attn, w_fin, b_fin,
               f_cam, f_pt, a_cam, a_pt):
    dn = (((1,), (1,)), ((), ()))
    cam_agg = cam_parts[0] + cam_parts[1]
    pt_agg = pt_parts[0] + pt_parts[1]
    h_cam = lax.dot_general(cam_agg, w_cam[...], dn, preferred_element_type=_F32)
    h_pt = lax.dot_general(pt_agg, w_point[...], dn, preferred_element_type=_F32)
    wf = w_fin[...]
    f_cam[...] = lax.dot_general(h_cam, wf[:, :_D], dn,
                                 preferred_element_type=_F32) + b_fin[...]
    f_pt[...] = lax.dot_general(h_pt, wf[:, _D:], dn, preferred_element_type=_F32)
    wa = w_attn[...]
    a_cam[...] = lax.dot_general(h_cam, wa[:, :_D], dn, preferred_element_type=_F32)
    a_pt[...] = lax.dot_general(h_pt, wa[:, _D:], dn, preferred_element_type=_F32)


_proj = pl.pallas_call(
    _proj_body,
    out_shape=(
        jax.ShapeDtypeStruct((_CAMS, _D), _F32),
        jax.ShapeDtypeStruct((_PTS, _D), _F32),
        jax.ShapeDtypeStruct((_CAMS, 1), _F32),
        jax.ShapeDtypeStruct((_PTS, 1), _F32),
    ),
)


@functools.partial(
    pl.kernel,
    out_type=(
        jax.ShapeDtypeStruct((_NW * _L,), _F32),
        jax.ShapeDtypeStruct((_NW * _L,), _F32),
    ),
    mesh=_mesh,
    compiler_params=pltpu.CompilerParams(needs_layout_passes=False),
    scratch_types=[
        pltpu.VMEM((_CAMS,), _F32),
        pltpu.VMEM((_PTS,), _F32),
        pltpu.VMEM((_EPW,), jnp.int32),
        pltpu.VMEM((_EPW,), jnp.int32),
        pltpu.VMEM((_EPW,), _F32),
        pltpu.VMEM((_L,), _F32),
        pltpu.VMEM((_L,), _F32),
    ],
)
def _stats(ci, pi, a_cam, a_pt, max_out, sum_out,
           ac_v, ap_v, ci_v, pi_v, lg_v, mx_v, sm_v):
    c = lax.axis_index("c")
    s = lax.axis_index("s")
    wid = c * _NS + s
    base = wid * _EPW
    pltpu.sync_copy(a_cam, ac_v)
    pltpu.sync_copy(a_pt, ap_v)
    pltpu.sync_copy(ci.at[pl.ds(base, _EPW)], ci_v)
    pltpu.sync_copy(pi.at[pl.ds(base, _EPW)], pi_v)

    @pl.loop(0, _EPW // _L, init_carry=jnp.full((_L,), -1e30, _F32))
    def mx(g, m):
        civ = ci_v[pl.ds(g * _L, _L)]
        piv = pi_v[pl.ds(g * _L, _L)]
        lg = plsc.load_gather(ac_v, [civ]) + plsc.load_gather(ap_v, [piv])
        lg_v[pl.ds(g * _L, _L)] = lg
        return jnp.maximum(m, lg)

    @pl.loop(0, _EPW // _L, init_carry=jnp.zeros((_L,), _F32))
    def sm(g, acc):
        return acc + jnp.exp(lg_v[pl.ds(g * _L, _L)] - mx)

    mx_v[...] = mx
    sm_v[...] = sm
    pltpu.sync_copy(mx_v, max_out.at[pl.ds(wid * _L, _L)])
    pltpu.sync_copy(sm_v, sum_out.at[pl.ds(wid * _L, _L)])


@functools.partial(
    pl.kernel,
    out_type=jax.ShapeDtypeStruct((_E, _D), _F32),
    mesh=_mesh,
    compiler_params=pltpu.CompilerParams(needs_layout_passes=False),
    scratch_types=(
        [pltpu.VMEM((_CH,), jnp.int32)] * (2 * _NBUF)
        + [pltpu.VMEM((_CH, _D), _F32)] * (2 * _NBUF)
        + [pltpu.VMEM((_CAMS,), _F32),
           pltpu.VMEM((_PTS,), _F32),
           pltpu.VMEM((_NW * _L,), _F32),
           pltpu.VMEM((_NW * _L,), _F32),
           pltpu.VMEM((_CH,), _F32)]
        + [pltpu.SemaphoreType.DMA] * (3 * _NBUF)
    ),
)
def _final(ci, pi, f_cam, f_pt, a_cam, a_pt, maxes, sums, out, *refs):
    civ = refs[0:_NBUF]
    piv = refs[_NBUF:2 * _NBUF]
    fcv = refs[2 * _NBUF:3 * _NBUF]
    fpv = refs[3 * _NBUF:4 * _NBUF]
    ac_v, ap_v, mx_b, sm_b, sc_v = refs[4 * _NBUF:4 * _NBUF + 5]
    sem_i = refs[4 * _NBUF + 5:5 * _NBUF + 5]
    sem_g = refs[5 * _NBUF + 5:6 * _NBUF + 5]
    sem_o = refs[6 * _NBUF + 5:7 * _NBUF + 5]
    c = lax.axis_index("c")
    s = lax.axis_index("s")
    wid = c * _NS + s
    base0 = wid * _EPW

    def start_idx(b, k):
        base = base0 + k * _CH
        pltpu.async_copy(ci.at[pl.ds(base, _CH)], civ[b], sem_i[b])
        pltpu.async_copy(pi.at[pl.ds(base, _CH)], piv[b], sem_i[b])

    def wait_idx(b):
        pltpu.make_async_copy(ci.at[pl.ds(0, _CH)], civ[b], sem_i[b]).wait()
        pltpu.make_async_copy(pi.at[pl.ds(0, _CH)], piv[b], sem_i[b]).wait()

    def fire_gathers(b):
        pltpu.async_copy(f_cam.at[civ[b]], fcv[b], sem_g[b])
        pltpu.async_copy(f_pt.at[piv[b]], fpv[b], sem_g[b])

    def wait_gathers(b):
        pltpu.make_async_copy(f_cam.at[civ[b]], fcv[b], sem_g[b]).wait()
        pltpu.make_async_copy(f_pt.at[piv[b]], fpv[b], sem_g[b]).wait()

    def wait_out(b):
        pltpu.make_async_copy(fcv[b], out.at[pl.ds(0, _CH)], sem_o[b]).wait()

    for b in range(_NBUF):
        start_idx(b, b)

    pltpu.sync_copy(maxes, mx_b)
    pltpu.sync_copy(sums, sm_b)
    pltpu.sync_copy(a_cam, ac_v)
    pltpu.sync_copy(a_pt, ap_v)

    @pl.loop(0, _NW, init_carry=jnp.full((_L,), -1e30, _F32))
    def mvec(i, m):
        return jnp.maximum(m, mx_b[pl.ds(i * _L, _L)])

    m = jnp.max(mvec)

    @pl.loop(0, _NW, init_carry=jnp.zeros((_L,), _F32))
    def svec(i, acc):
        return acc + jnp.exp(mx_b[pl.ds(i * _L, _L)] - m) * sm_b[pl.ds(i * _L, _L)]

    inv_s = jnp.full((_L,), 1.0, _F32) / jnp.full((_L,), jnp.sum(svec), _F32)

    for b in range(2):
        wait_idx(b)
        fire_gathers(b)

    @pl.loop(0, _NCHUNK, step=_NBUF)
    def _chunk(k0):
        for b in range(_NBUF):
            k = k0 + b
            nb = (b + 2) % _NBUF

            @pl.when(k + 2 < _NCHUNK)
            def _():
                wait_idx(nb)

                @pl.when(k + 2 >= _NBUF)
                def _():
                    wait_out(nb)

                fire_gathers(nb)

            for g in range(_CH // _L):
                cg = civ[b][pl.ds(g * _L, _L)]
                pg = piv[b][pl.ds(g * _L, _L)]
                lg = plsc.load_gather(ac_v, [cg]) + plsc.load_gather(ap_v, [pg])
                sc_v[pl.ds(g * _L, _L)] = jnp.exp(lg - m) * inv_s

            wait_gathers(b)

            @pl.loop(0, _CH // _L)
            def _grp(g):
                e0 = g * _L
                for lane in range(_L):
                    e = e0 + lane
                    sb = plsc.load_gather(sc_v, [jnp.full((_L,), e, jnp.int32)])
                    for j in range(_D // _L):
                        v = (fcv[b][e, pl.ds(j * _L, _L)]
                             + fpv[b][e, pl.ds(j * _L, _L)]) * sb
                        fcv[b][e, pl.ds(j * _L, _L)] = jnp.maximum(v, 0.0)

            pltpu.async_copy(fcv[b], out.at[pl.ds(base0 + k * _CH, _CH)], sem_o[b])

            @pl.when(k + _NBUF < _NCHUNK)
            def _():
                start_idx(b, k + _NBUF)

    for b in range(_NBUF):
        wait_out(b)


def kernel(edge_values, cam_indices, point_indices, W_cam, W_point, W_attn,
           W_fin, b_fin):
    ci = cam_indices.astype(jnp.int32)
    pi = point_indices.astype(jnp.int32)
    cam_parts, pt_parts = _seg_sum(edge_values, ci, pi)
    f_cam, f_pt, a_cam2, a_pt2 = _proj(
        cam_parts, pt_parts, W_cam, W_point, W_attn, W_fin,
        b_fin.reshape(1, _D))
    a_cam = a_cam2.reshape(_CAMS)
    a_pt = a_pt2.reshape(_PTS)
    mx, sm = _stats(ci, pi, a_cam, a_pt)
    return _final(ci, pi, f_cam, f_pt, a_cam, a_pt, mx, sm)


# speedup vs baseline: 1.0194x; 1.0194x over previous
import functools

import jax
import jax.numpy as jnp
from jax import lax
from jax.experimental import pallas as pl
from jax.experimental.pallas import tpu as pltpu
from jax.experimental.pallas import tpu_sc as plsc

_NC, _NS, _L = 2, 16, 16
_NW = _NC * _NS
_E = 320000
_EPW = _E // _NW
_CH = 80
_NCHUNK = _EPW // _CH
_CHA = 40
_NCHA = _EPW // _CHA
_CAMS, _PTS, _D = 2000, 8000, 128
_F32 = jnp.float32

_mesh = plsc.VectorSubcoreMesh(
    core_axis_name="c", subcore_axis_name="s", num_cores=_NC, num_subcores=_NS
)


_NBUF = 5
_NBA = 8

@functools.partial(
    pl.kernel,
    out_type=(
        jax.ShapeDtypeStruct((_NC, _CAMS, _D), _F32),
        jax.ShapeDtypeStruct((_NC, _PTS, _D), _F32),
    ),
    mesh=_mesh,
    compiler_params=pltpu.CompilerParams(needs_layout_passes=False),
    scratch_types=(
        [pltpu.VMEM((_CHA, _D), _F32)] * _NBA
        + [pltpu.VMEM((_CHA,), jnp.int32)] * (2 * _NBA)
        + [pltpu.VMEM((16, _D), _F32),
           pltpu.VMEM_SHARED((_CAMS, _D), _F32),
           pltpu.VMEM_SHARED((_PTS, _D), _F32)]
        + [pltpu.SemaphoreType.DMA] * (2 * _NBA)
    ),
)
def _seg_sum(edges, ci, pi, cam_out, pt_out, *refs):
    rows = refs[0:_NBA]
    civ = refs[_NBA:2 * _NBA]
    piv = refs[2 * _NBA:3 * _NBA]
    zbuf, cam_sh, pt_sh = refs[3 * _NBA:3 * _NBA + 3]
    sem_in = refs[3 * _NBA + 3:4 * _NBA + 3]
    sem_sc = refs[4 * _NBA + 3:5 * _NBA + 3]
    c = lax.axis_index("c")
    s = lax.axis_index("s")
    zeros16 = jnp.zeros((_L,), _F32)

    wid = c * _NS + s
    base0 = wid * _EPW

    def start_in(b, k):
        base = base0 + k * _CHA
        pltpu.async_copy(edges.at[pl.ds(base, _CHA)], rows[b], sem_in[b])
        pltpu.async_copy(ci.at[pl.ds(base, _CHA)], civ[b], sem_in[b])
        pltpu.async_copy(pi.at[pl.ds(base, _CHA)], piv[b], sem_in[b])

    def wait_in(b):
        pltpu.make_async_copy(edges.at[pl.ds(0, _CHA)], rows[b], sem_in[b]).wait()
        pltpu.make_async_copy(ci.at[pl.ds(0, _CHA)], civ[b], sem_in[b]).wait()
        pltpu.make_async_copy(pi.at[pl.ds(0, _CHA)], piv[b], sem_in[b]).wait()

    def wait_scat(b):
        pltpu.make_async_copy(rows[b], cam_sh.at[civ[b]], sem_sc[b]).wait()
        pltpu.make_async_copy(rows[b], pt_sh.at[piv[b]], sem_sc[b]).wait()

    for b in range(_NBA):
        start_in(b, b)

    @pl.loop(0, 16)
    def _zero(i):
        for j in range(_D // _L):
            zbuf[i, pl.ds(j * _L, _L)] = zeros16

    @pl.loop(0, _CAMS // 16)
    def _zc(b):
        @pl.when(lax.rem(b, _NS) == s)
        def _():
            pltpu.sync_copy(zbuf, cam_sh.at[pl.ds(b * 16, 16)])

    @pl.loop(0, _PTS // 16)
    def _zp(b):
        @pl.when(lax.rem(b, _NS) == s)
        def _():
            pltpu.sync_copy(zbuf, pt_sh.at[pl.ds(b * 16, 16)])

    plsc.subcore_barrier()

    def visit(k, b):
        wait_in(b)
        pltpu.async_copy(rows[b], cam_sh.at[civ[b]], sem_sc[b], add=True)
        pltpu.async_copy(rows[b], pt_sh.at[piv[b]], sem_sc[b], add=True)
        pb = (b - 2) % _NBA

        @pl.when(k > 1)
        def _():
            wait_scat(pb)

            @pl.when(k - 2 + _NBA < _NCHA)
            def _():
                start_in(pb, k - 2 + _NBA)

    @pl.loop(0, _NCHA - 2, step=_NBA)
    def _chunk(k0):
        for b in range(_NBA):
            visit(k0 + b, b)

    visit(_NCHA - 2, (_NCHA - 2) % _NBA)
    visit(_NCHA - 1, (_NCHA - 1) % _NBA)
    wait_scat((_NCHA - 2) % _NBA)
    wait_scat((_NCHA - 1) % _NBA)
    plsc.subcore_barrier()

    @pl.loop(0, _CAMS // 16)
    def _wc(b):
        @pl.when(lax.rem(b, _NS) == s)
        def _():
            pltpu.sync_copy(cam_sh.at[pl.ds(b * 16, 16)],
                            cam_out.at[c, pl.ds(b * 16, 16)])

    @pl.loop(0, _PTS // 16)
    def _wp(b):
        @pl.when(lax.rem(b, _NS) == s)
        def _():
            pltpu.sync_copy(pt_sh.at[pl.ds(b * 16, 16)],
                            pt_out.at[c, pl.ds(b * 16, 16)])


def _proj_body(cam_parts, pt_parts, w_cam, w_point, w_attn, w_fin, b_fin,
               f_cam, f_pt, a_cam, a_pt):
    dn = (((1,), (1,)), ((), ()))
    cam_agg = cam_parts[0] + cam_parts[1]
    pt_agg = pt_parts[0] + pt_parts[1]
    h_cam = lax.dot_general(cam_agg, w_cam[...], dn, preferred_element_type=_F32)
    h_pt = lax.dot_general(pt_agg, w_point[...], dn, preferred_element_type=_F32)
    wf = w_fin[...]
    f_cam[...] = lax.dot_general(h_cam, wf[:, :_D], dn,
                                 preferred_element_type=_F32) + b_fin[...]
    f_pt[...] = lax.dot_general(h_pt, wf[:, _D:], dn, preferred_element_type=_F32)
    wa = w_attn[...]
    a_cam[...] = lax.dot_general(h_cam, wa[:, :_D], dn, preferred_element_type=_F32)
    a_pt[...] = lax.dot_general(h_pt, wa[:, _D:], dn, preferred_element_type=_F32)


_proj = pl.pallas_call(
    _proj_body,
    out_shape=(
        jax.ShapeDtypeStruct((_CAMS, _D), _F32),
        jax.ShapeDtypeStruct((_PTS, _D), _F32),
        jax.ShapeDtypeStruct((_CAMS, 1), _F32),
        jax.ShapeDtypeStruct((_PTS, 1), _F32),
    ),
)


@functools.partial(
    pl.kernel,
    out_type=(
        jax.ShapeDtypeStruct((_NW * _L,), _F32),
        jax.ShapeDtypeStruct((_NW * _L,), _F32),
    ),
    mesh=_mesh,
    compiler_params=pltpu.CompilerParams(needs_layout_passes=False),
    scratch_types=[
        pltpu.VMEM((_CAMS,), _F32),
        pltpu.VMEM((_PTS,), _F32),
        pltpu.VMEM((_EPW,), jnp.int32),
        pltpu.VMEM((_EPW,), jnp.int32),
        pltpu.VMEM((_EPW,), _F32),
        pltpu.VMEM((_L,), _F32),
        pltpu.VMEM((_L,), _F32),
    ],
)
def _stats(ci, pi, a_cam, a_pt, max_out, sum_out,
           ac_v, ap_v, ci_v, pi_v, lg_v, mx_v, sm_v):
    c = lax.axis_index("c")
    s = lax.axis_index("s")
    wid = c * _NS + s
    base = wid * _EPW
    pltpu.sync_copy(a_cam, ac_v)
    pltpu.sync_copy(a_pt, ap_v)
    pltpu.sync_copy(ci.at[pl.ds(base, _EPW)], ci_v)
    pltpu.sync_copy(pi.at[pl.ds(base, _EPW)], pi_v)

    @pl.loop(0, _EPW // _L, init_carry=jnp.full((_L,), -1e30, _F32))
    def mx(g, m):
        civ = ci_v[pl.ds(g * _L, _L)]
        piv = pi_v[pl.ds(g * _L, _L)]
        lg = plsc.load_gather(ac_v, [civ]) + plsc.load_gather(ap_v, [piv])
        lg_v[pl.ds(g * _L, _L)] = lg
        return jnp.maximum(m, lg)

    @pl.loop(0, _EPW // _L, init_carry=jnp.zeros((_L,), _F32))
    def sm(g, acc):
        return acc + jnp.exp(lg_v[pl.ds(g * _L, _L)] - mx)

    mx_v[...] = mx
    sm_v[...] = sm
    pltpu.sync_copy(mx_v, max_out.at[pl.ds(wid * _L, _L)])
    pltpu.sync_copy(sm_v, sum_out.at[pl.ds(wid * _L, _L)])


@functools.partial(
    pl.kernel,
    out_type=jax.ShapeDtypeStruct((_E, _D), _F32),
    mesh=_mesh,
    compiler_params=pltpu.CompilerParams(needs_layout_passes=False),
    scratch_types=(
        [pltpu.VMEM((_CH,), jnp.int32)] * (2 * _NBUF)
        + [pltpu.VMEM((_CH, _D), _F32)] * (2 * _NBUF)
        + [pltpu.VMEM((_CAMS,), _F32),
           pltpu.VMEM((_PTS,), _F32),
           pltpu.VMEM((_NW * _L,), _F32),
           pltpu.VMEM((_NW * _L,), _F32),
           pltpu.VMEM((_CH,), _F32)]
        + [pltpu.SemaphoreType.DMA] * (3 * _NBUF)
    ),
)
def _final(ci, pi, f_cam, f_pt, a_cam, a_pt, maxes, sums, out, *refs):
    civ = refs[0:_NBUF]
    piv = refs[_NBUF:2 * _NBUF]
    fcv = refs[2 * _NBUF:3 * _NBUF]
    fpv = refs[3 * _NBUF:4 * _NBUF]
    ac_v, ap_v, mx_b, sm_b, sc_v = refs[4 * _NBUF:4 * _NBUF + 5]
    sem_i = refs[4 * _NBUF + 5:5 * _NBUF + 5]
    sem_g = refs[5 * _NBUF + 5:6 * _NBUF + 5]
    sem_o = refs[6 * _NBUF + 5:7 * _NBUF + 5]
    c = lax.axis_index("c")
    s = lax.axis_index("s")
    wid = c * _NS + s
    base0 = wid * _EPW

    def start_idx(b, k):
        base = base0 + k * _CH
        pltpu.async_copy(ci.at[pl.ds(base, _CH)], civ[b], sem_i[b])
        pltpu.async_copy(pi.at[pl.ds(base, _CH)], piv[b], sem_i[b])

    def wait_idx(b):
        pltpu.make_async_copy(ci.at[pl.ds(0, _CH)], civ[b], sem_i[b]).wait()
        pltpu.make_async_copy(pi.at[pl.ds(0, _CH)], piv[b], sem_i[b]).wait()

    def fire_gathers(b):
        pltpu.async_copy(f_cam.at[civ[b]], fcv[b], sem_g[b])
        pltpu.async_copy(f_pt.at[piv[b]], fpv[b], sem_g[b])

    def wait_gathers(b):
        pltpu.make_async_copy(f_cam.at[civ[b]], fcv[b], sem_g[b]).wait()
        pltpu.make_async_copy(f_pt.at[piv[b]], fpv[b], sem_g[b]).wait()

    def wait_out(b):
        pltpu.make_async_copy(fcv[b], out.at[pl.ds(0, _CH)], sem_o[b]).wait()

    for b in range(_NBUF):
        start_idx(b, b)

    pltpu.sync_copy(maxes, mx_b)
    pltpu.sync_copy(sums, sm_b)
    pltpu.sync_copy(a_cam, ac_v)
    pltpu.sync_copy(a_pt, ap_v)

    @pl.loop(0, _NW, init_carry=jnp.full((_L,), -1e30, _F32))
    def mvec(i, m):
        return jnp.maximum(m, mx_b[pl.ds(i * _L, _L)])

    m = jnp.max(mvec)

    @pl.loop(0, _NW, init_carry=jnp.zeros((_L,), _F32))
    def svec(i, acc):
        return acc + jnp.exp(mx_b[pl.ds(i * _L, _L)] - m) * sm_b[pl.ds(i * _L, _L)]

    inv_s = jnp.full((_L,), 1.0, _F32) / jnp.full((_L,), jnp.sum(svec), _F32)

    for b in range(2):
        wait_idx(b)
        fire_gathers(b)

    @pl.loop(0, _NCHUNK, step=_NBUF)
    def _chunk(k0):
        for b in range(_NBUF):
            k = k0 + b
            nb = (b + 2) % _NBUF

            @pl.when(k + 2 < _NCHUNK)
            def _():
                wait_idx(nb)

                @pl.when(k + 2 >= _NBUF)
                def _():
                    wait_out(nb)

                fire_gathers(nb)

            for g in range(_CH // _L):
                cg = civ[b][pl.ds(g * _L, _L)]
                pg = piv[b][pl.ds(g * _L, _L)]
                lg = plsc.load_gather(ac_v, [cg]) + plsc.load_gather(ap_v, [pg])
                sc_v[pl.ds(g * _L, _L)] = jnp.exp(lg - m) * inv_s

            wait_gathers(b)

            @pl.loop(0, _CH // _L)
            def _grp(g):
                e0 = g * _L
                for lane in range(_L):
                    e = e0 + lane
                    sb = plsc.load_gather(sc_v, [jnp.full((_L,), e, jnp.int32)])
                    for j in range(_D // _L):
                        v = (fcv[b][e, pl.ds(j * _L, _L)]
                             + fpv[b][e, pl.ds(j * _L, _L)]) * sb
                        fcv[b][e, pl.ds(j * _L, _L)] = jnp.maximum(v, 0.0)

            pltpu.async_copy(fcv[b], out.at[pl.ds(base0 + k * _CH, _CH)], sem_o[b])

            @pl.when(k + _NBUF < _NCHUNK)
            def _():
                start_idx(b, k + _NBUF)

    for b in range(_NBUF):
        wait_out(b)


def kernel(edge_values, cam_indices, point_indices, W_cam, W_point, W_attn,
           W_fin, b_fin):
    ci = cam_indices.astype(jnp.int32)
    pi = point_indices.astype(jnp.int32)
    cam_parts, pt_parts = _seg_sum(edge_values, ci, pi)
    f_cam, f_pt, a_cam2, a_pt2 = _proj(
        cam_parts, pt_parts, W_cam, W_point, W_attn, W_fin,
        b_fin.reshape(1, _D))
    a_cam = a_cam2.reshape(_CAMS)
    a_pt = a_pt2.reshape(_PTS)
    mx, sm = _stats(ci, pi, a_cam, a_pt)
    return _final(ci, pi, f_cam, f_pt, a_cam, a_pt, mx, sm)


# speedup vs baseline: 1.0332x; 1.0136x over previous
import functools

import jax
import jax.numpy as jnp
from jax import lax
from jax.experimental import pallas as pl
from jax.experimental.pallas import tpu as pltpu
from jax.experimental.pallas import tpu_sc as plsc

_NC, _NS, _L = 2, 16, 16
_NW = _NC * _NS
_E = 320000
_EPW = _E // _NW
_CH = 80
_NCHUNK = _EPW // _CH
_CHA = 40
_NCHA = _EPW // _CHA
_CAMS, _PTS, _D = 2000, 8000, 128
_F32 = jnp.float32

_mesh = plsc.VectorSubcoreMesh(
    core_axis_name="c", subcore_axis_name="s", num_cores=_NC, num_subcores=_NS
)


_NBUF = 5

@functools.partial(
    pl.kernel,
    out_type=(
        jax.ShapeDtypeStruct((_NC, _CAMS, _D), _F32),
        jax.ShapeDtypeStruct((_NC, _PTS, _D), _F32),
    ),
    mesh=_mesh,
    compiler_params=pltpu.CompilerParams(needs_layout_passes=False),
    scratch_types=(
        [pltpu.VMEM((_CHA, _D), _F32)] * _NBUF
        + [pltpu.VMEM((_CHA,), jnp.int32)] * (2 * _NBUF)
        + [pltpu.VMEM((16, _D), _F32),
           pltpu.VMEM_SHARED((_CAMS, _D), _F32),
           pltpu.VMEM_SHARED((_PTS, _D), _F32)]
        + [pltpu.SemaphoreType.DMA] * (2 * _NBUF)
    ),
)
def _seg_sum(edges, ci, pi, cam_out, pt_out, *refs):
    rows = refs[0:_NBUF]
    civ = refs[_NBUF:2 * _NBUF]
    piv = refs[2 * _NBUF:3 * _NBUF]
    zbuf, cam_sh, pt_sh = refs[3 * _NBUF:3 * _NBUF + 3]
    sem_in = refs[3 * _NBUF + 3:4 * _NBUF + 3]
    sem_sc = refs[4 * _NBUF + 3:5 * _NBUF + 3]
    c = lax.axis_index("c")
    s = lax.axis_index("s")
    zeros16 = jnp.zeros((_L,), _F32)

    wid = c * _NS + s
    base0 = wid * _EPW

    def start_in(b, k):
        base = base0 + k * _CHA
        pltpu.async_copy(edges.at[pl.ds(base, _CHA)], rows[b], sem_in[b])
        pltpu.async_copy(ci.at[pl.ds(base, _CHA)], civ[b], sem_in[b])
        pltpu.async_copy(pi.at[pl.ds(base, _CHA)], piv[b], sem_in[b])

    def wait_in(b):
        pltpu.make_async_copy(edges.at[pl.ds(0, _CHA)], rows[b], sem_in[b]).wait()
        pltpu.make_async_copy(ci.at[pl.ds(0, _CHA)], civ[b], sem_in[b]).wait()
        pltpu.make_async_copy(pi.at[pl.ds(0, _CHA)], piv[b], sem_in[b]).wait()

    def wait_scat(b):
        pltpu.make_async_copy(rows[b], cam_sh.at[civ[b]], sem_sc[b]).wait()
        pltpu.make_async_copy(rows[b], pt_sh.at[piv[b]], sem_sc[b]).wait()

    for b in range(_NBUF):
        start_in(b, b)

    @pl.loop(0, 16)
    def _zero(i):
        for j in range(_D // _L):
            zbuf[i, pl.ds(j * _L, _L)] = zeros16

    @pl.loop(0, _CAMS // 16)
    def _zc(b):
        @pl.when(lax.rem(b, _NS) == s)
        def _():
            pltpu.sync_copy(zbuf, cam_sh.at[pl.ds(b * 16, 16)])

    @pl.loop(0, _PTS // 16)
    def _zp(b):
        @pl.when(lax.rem(b, _NS) == s)
        def _():
            pltpu.sync_copy(zbuf, pt_sh.at[pl.ds(b * 16, 16)])

    plsc.subcore_barrier()

    @pl.loop(0, _NCHA, step=_NBUF)
    def _chunk(k0):
        for b in range(_NBUF):
            k = k0 + b
            wait_in(b)
            pltpu.async_copy(rows[b], cam_sh.at[civ[b]], sem_sc[b], add=True)
            pltpu.async_copy(rows[b], pt_sh.at[piv[b]], sem_sc[b], add=True)
            pb = (b - 2) % _NBUF

            @pl.when(k > 1)
            def _():
                wait_scat(pb)

                @pl.when(k - 2 + _NBUF < _NCHA)
                def _():
                    start_in(pb, k - 2 + _NBUF)

    wait_scat((_NCHA - 2) % _NBUF)
    wait_scat((_NCHA - 1) % _NBUF)
    plsc.subcore_barrier()

    @pl.loop(0, _CAMS // 16)
    def _wc(b):
        @pl.when(lax.rem(b, _NS) == s)
        def _():
            pltpu.sync_copy(cam_sh.at[pl.ds(b * 16, 16)],
                            cam_out.at[c, pl.ds(b * 16, 16)])

    @pl.loop(0, _PTS // 16)
    def _wp(b):
        @pl.when(lax.rem(b, _NS) == s)
        def _():
            pltpu.sync_copy(pt_sh.at[pl.ds(b * 16, 16)],
                            pt_out.at[c, pl.ds(b * 16, 16)])


def _proj_body(cam_parts, pt_parts, w_cam, w_point, w_attn, w_fin, b_fin,
               f_cam, f_pt, a_cam, a_pt):
    dn = (((1,), (1,)), ((), ()))
    cam_agg = cam_parts[0] + cam_parts[1]
    pt_agg = pt_parts[0] + pt_parts[1]
    h_cam = lax.dot_general(cam_agg, w_cam[...], dn, preferred_element_type=_F32)
    h_pt = lax.dot_general(pt_agg, w_point[...], dn, preferred_element_type=_F32)
    wf = w_fin[...]
    f_cam[...] = lax.dot_general(h_cam, wf[:, :_D], dn,
                                 preferred_element_type=_F32) + b_fin[...]
    f_pt[...] = lax.dot_general(h_pt, wf[:, _D:], dn, preferred_element_type=_F32)
    wa = w_attn[...]
    a_cam[...] = lax.dot_general(h_cam, wa[:, :_D], dn, preferred_element_type=_F32)
    a_pt[...] = lax.dot_general(h_pt, wa[:, _D:], dn, preferred_element_type=_F32)


_proj = pl.pallas_call(
    _proj_body,
    out_shape=(
        jax.ShapeDtypeStruct((_CAMS, _D), _F32),
        jax.ShapeDtypeStruct((_PTS, _D), _F32),
        jax.ShapeDtypeStruct((_CAMS, 1), _F32),
        jax.ShapeDtypeStruct((_PTS, 1), _F32),
    ),
)


@functools.partial(
    pl.kernel,
    out_type=(
        jax.ShapeDtypeStruct((_NW * _L,), _F32),
        jax.ShapeDtypeStruct((_NW * _L,), _F32),
        jax.ShapeDtypeStruct((_E,), _F32),
    ),
    mesh=_mesh,
    compiler_params=pltpu.CompilerParams(needs_layout_passes=False),
    scratch_types=[
        pltpu.VMEM((_CAMS,), _F32),
        pltpu.VMEM((_PTS,), _F32),
        pltpu.VMEM((_EPW,), jnp.int32),
        pltpu.VMEM((_EPW,), jnp.int32),
        pltpu.VMEM((_EPW,), _F32),
        pltpu.VMEM((_L,), _F32),
        pltpu.VMEM((_L,), _F32),
    ],
)
def _stats(ci, pi, a_cam, a_pt, max_out, sum_out, lg_out,
           ac_v, ap_v, ci_v, pi_v, lg_v, mx_v, sm_v):
    c = lax.axis_index("c")
    s = lax.axis_index("s")
    wid = c * _NS + s
    base = wid * _EPW
    pltpu.sync_copy(a_cam, ac_v)
    pltpu.sync_copy(a_pt, ap_v)
    pltpu.sync_copy(ci.at[pl.ds(base, _EPW)], ci_v)
    pltpu.sync_copy(pi.at[pl.ds(base, _EPW)], pi_v)

    @pl.loop(0, _EPW // _L, init_carry=jnp.full((_L,), -1e30, _F32))
    def mx(g, m):
        civ = ci_v[pl.ds(g * _L, _L)]
        piv = pi_v[pl.ds(g * _L, _L)]
        lg = plsc.load_gather(ac_v, [civ]) + plsc.load_gather(ap_v, [piv])
        lg_v[pl.ds(g * _L, _L)] = lg
        return jnp.maximum(m, lg)

    @pl.loop(0, _EPW // _L, init_carry=jnp.zeros((_L,), _F32))
    def sm(g, acc):
        return acc + jnp.exp(lg_v[pl.ds(g * _L, _L)] - mx)

    mx_v[...] = mx
    sm_v[...] = sm
    pltpu.sync_copy(lg_v, lg_out.at[pl.ds(base, _EPW)])
    pltpu.sync_copy(mx_v, max_out.at[pl.ds(wid * _L, _L)])
    pltpu.sync_copy(sm_v, sum_out.at[pl.ds(wid * _L, _L)])


@functools.partial(
    pl.kernel,
    out_type=jax.ShapeDtypeStruct((_E, _D), _F32),
    mesh=_mesh,
    compiler_params=pltpu.CompilerParams(needs_layout_passes=False),
    scratch_types=(
        [pltpu.VMEM((_CH,), jnp.int32)] * (2 * _NBUF)
        + [pltpu.VMEM((_CH,), _F32)] * _NBUF
        + [pltpu.VMEM((_CH, _D), _F32)] * (2 * _NBUF)
        + [pltpu.VMEM((_NW * _L,), _F32),
           pltpu.VMEM((_NW * _L,), _F32),
           pltpu.VMEM((_CH,), _F32)]
        + [pltpu.SemaphoreType.DMA] * (3 * _NBUF)
    ),
)
def _final(ci, pi, f_cam, f_pt, lgs, maxes, sums, out, *refs):
    civ = refs[0:_NBUF]
    piv = refs[_NBUF:2 * _NBUF]
    lgv = refs[2 * _NBUF:3 * _NBUF]
    fcv = refs[3 * _NBUF:4 * _NBUF]
    fpv = refs[4 * _NBUF:5 * _NBUF]
    mx_b, sm_b, sc_v = refs[5 * _NBUF:5 * _NBUF + 3]
    sem_i = refs[5 * _NBUF + 3:6 * _NBUF + 3]
    sem_g = refs[6 * _NBUF + 3:7 * _NBUF + 3]
    sem_o = refs[7 * _NBUF + 3:8 * _NBUF + 3]
    c = lax.axis_index("c")
    s = lax.axis_index("s")
    wid = c * _NS + s
    base0 = wid * _EPW

    def start_idx(b, k):
        base = base0 + k * _CH
        pltpu.async_copy(ci.at[pl.ds(base, _CH)], civ[b], sem_i[b])
        pltpu.async_copy(pi.at[pl.ds(base, _CH)], piv[b], sem_i[b])
        pltpu.async_copy(lgs.at[pl.ds(base, _CH)], lgv[b], sem_i[b])

    def wait_idx(b):
        pltpu.make_async_copy(ci.at[pl.ds(0, _CH)], civ[b], sem_i[b]).wait()
        pltpu.make_async_copy(pi.at[pl.ds(0, _CH)], piv[b], sem_i[b]).wait()
        pltpu.make_async_copy(lgs.at[pl.ds(0, _CH)], lgv[b], sem_i[b]).wait()

    def fire_gathers(b):
        pltpu.async_copy(f_cam.at[civ[b]], fcv[b], sem_g[b])
        pltpu.async_copy(f_pt.at[piv[b]], fpv[b], sem_g[b])

    def wait_gathers(b):
        pltpu.make_async_copy(f_cam.at[civ[b]], fcv[b], sem_g[b]).wait()
        pltpu.make_async_copy(f_pt.at[piv[b]], fpv[b], sem_g[b]).wait()

    def wait_out(b):
        pltpu.make_async_copy(fcv[b], out.at[pl.ds(0, _CH)], sem_o[b]).wait()

    for b in range(_NBUF):
        start_idx(b, b)

    pltpu.sync_copy(maxes, mx_b)
    pltpu.sync_copy(sums, sm_b)

    @pl.loop(0, _NW, init_carry=jnp.full((_L,), -1e30, _F32))
    def mvec(i, m):
        return jnp.maximum(m, mx_b[pl.ds(i * _L, _L)])

    m = jnp.max(mvec)

    @pl.loop(0, _NW, init_carry=jnp.zeros((_L,), _F32))
    def svec(i, acc):
        return acc + jnp.exp(mx_b[pl.ds(i * _L, _L)] - m) * sm_b[pl.ds(i * _L, _L)]

    inv_s = jnp.full((_L,), 1.0, _F32) / jnp.full((_L,), jnp.sum(svec), _F32)

    for b in range(2):
        wait_idx(b)
        fire_gathers(b)

    @pl.loop(0, _NCHUNK, step=_NBUF)
    def _chunk(k0):
        for b in range(_NBUF):
            k = k0 + b
            nb = (b + 2) % _NBUF

            @pl.when(k + 2 < _NCHUNK)
            def _():
                wait_idx(nb)

                @pl.when(k + 2 >= _NBUF)
                def _():
                    wait_out(nb)

                fire_gathers(nb)

            for g in range(_CH // _L):
                lg = lgv[b][pl.ds(g * _L, _L)]
                sc_v[pl.ds(g * _L, _L)] = jnp.exp(lg - m) * inv_s

            wait_gathers(b)

            @pl.loop(0, _CH // _L)
            def _grp(g):
                e0 = g * _L
                for lane in range(_L):
                    e = e0 + lane
                    sb = plsc.load_gather(sc_v, [jnp.full((_L,), e, jnp.int32)])
                    for j in range(_D // _L):
                        v = (fcv[b][e, pl.ds(j * _L, _L)]
                             + fpv[b][e, pl.ds(j * _L, _L)]) * sb
                        fcv[b][e, pl.ds(j * _L, _L)] = jnp.maximum(v, 0.0)

            pltpu.async_copy(fcv[b], out.at[pl.ds(base0 + k * _CH, _CH)], sem_o[b])

            @pl.when(k + _NBUF < _NCHUNK)
            def _():
                start_idx(b, k + _NBUF)

    for b in range(_NBUF):
        wait_out(b)


def kernel(edge_values, cam_indices, point_indices, W_cam, W_point, W_attn,
           W_fin, b_fin):
    ci = cam_indices.astype(jnp.int32)
    pi = point_indices.astype(jnp.int32)
    cam_parts, pt_parts = _seg_sum(edge_values, ci, pi)
    f_cam, f_pt, a_cam2, a_pt2 = _proj(
        cam_parts, pt_parts, W_cam, W_point, W_attn, W_fin,
        b_fin.reshape(1, _D))
    a_cam = a_cam2.reshape(_CAMS)
    a_pt = a_pt2.reshape(_PTS)
    mx, sm, lgs = _stats(ci, pi, a_cam, a_pt)
    return _final(ci, pi, f_cam, f_pt, lgs, mx, sm)
